# Initial kernel scaffold; baseline (speedup 1.0000x reference)
import functools

import jax
import jax.numpy as jnp
from jax import lax
from jax.experimental import pallas as pl
from jax.experimental.pallas import tpu as pltpu
from jax.experimental.pallas import tpu_sc as plsc

N = 10000
E = 320000
C = 128
D_EDGE = 4
D_OUT = 2

NC = 2
NS = 16
NW = NC * NS
K = 128
NCH = 79
EP = NW * NCH * K
EPW = EP // NW
NPAD = 10240
NSL = NPAD // NS

_f32 = jnp.float32
_i32 = jnp.int32


def _tc_xlxr(x, W_l, W_r):
    def body(x_ref, wl_ref, wr_ref, xl_ref, xr_ref):
        xb = x_ref[...]
        xl_ref[...] = jnp.dot(xb, wl_ref[...], preferred_element_type=_f32)
        xr_ref[...] = jnp.dot(xb, wr_ref[...], preferred_element_type=_f32)

    return pl.pallas_call(
        body,
        grid=(10,),
        in_specs=[
            pl.BlockSpec((1000, C), lambda i: (i, 0)),
            pl.BlockSpec((C, C), lambda i: (0, 0)),
            pl.BlockSpec((C, C), lambda i: (0, 0)),
        ],
        out_specs=[
            pl.BlockSpec((1000, C), lambda i: (i, 0)),
            pl.BlockSpec((1000, C), lambda i: (i, 0)),
        ],
        out_shape=[
            jax.ShapeDtypeStruct((N, C), _f32),
            jax.ShapeDtypeStruct((N, C), _f32),
        ],
    )(x, W_l, W_r)


def _tc_ea(efp, W_e):
    def body(ef_ref, we_ref, ea_ref):
        ea_ref[...] = jnp.dot(ef_ref[...], we_ref[...],
                              preferred_element_type=_f32)

    return pl.pallas_call(
        body,
        grid=(NCH,),
        in_specs=[
            pl.BlockSpec((EP // NCH, D_EDGE), lambda i: (i, 0)),
            pl.BlockSpec((D_EDGE, C), lambda i: (0, 0)),
        ],
        out_specs=pl.BlockSpec((EP // NCH, C), lambda i: (i, 0)),
        out_shape=jax.ShapeDtypeStruct((EP, C), _f32),
    )(efp, W_e)


def _tc_mid(hp, b_gat, W_gcn, degp):
    def body(hp_ref, bg_ref, wg_ref, dp_ref, hw_ref, dis_ref):
        h = jnp.maximum(hp_ref[0] + hp_ref[1] + bg_ref[...][None, :], 0.0)
        hw_ref[...] = jnp.dot(h, wg_ref[...], preferred_element_type=_f32)
        deg = dp_ref[0] + dp_ref[1]
        dis_ref[...] = jnp.where(
            deg > 0, lax.rsqrt(jnp.maximum(deg, 1e-12)), 0.0)

    return pl.pallas_call(
        body,
        grid=(10,),
        in_specs=[
            pl.BlockSpec((2, 1024, C), lambda i: (0, i, 0)),
            pl.BlockSpec((C,), lambda i: (0,)),
            pl.BlockSpec((C, C), lambda i: (0, 0)),
            pl.BlockSpec((2, 1024), lambda i: (0, i)),
        ],
        out_specs=[
            pl.BlockSpec((1024, C), lambda i: (i, 0)),
            pl.BlockSpec((1024,), lambda i: (i,)),
        ],
        out_shape=[
            jax.ShapeDtypeStruct((NPAD, C), _f32),
            jax.ShapeDtypeStruct((NPAD,), _f32),
        ],
    )(hp, b_gat, W_gcn, degp)


def _tc_out(h2p, b_gcn, W_out, b_out):
    def body(h2p_ref, bg_ref, wo_ref, bo_ref, out_ref):
        h2 = jnp.maximum(h2p_ref[0] + h2p_ref[1] + bg_ref[...][None, :], 0.0)
        out_ref[...] = (jnp.dot(h2, wo_ref[...], preferred_element_type=_f32)
                        + bo_ref[...][None, :])

    return pl.pallas_call(
        body,
        grid=(10,),
        in_specs=[
            pl.BlockSpec((2, 1024, C), lambda i: (0, i, 0)),
            pl.BlockSpec((C,), lambda i: (0,)),
            pl.BlockSpec((C, D_OUT), lambda i: (0, 0)),
            pl.BlockSpec((D_OUT,), lambda i: (0,)),
        ],
        out_specs=pl.BlockSpec((1024, D_OUT), lambda i: (i, 0)),
        out_shape=jax.ShapeDtypeStruct((NPAD, D_OUT), _f32),
    )(h2p, b_gcn, W_out, b_out)


def _sc_mesh():
    return plsc.VectorSubcoreMesh(core_axis_name="c", subcore_axis_name="s",
                                  num_cores=NC, num_subcores=NS)


def _sc_pass_a(src, dst, xl, xr, ea, attf, zn):
    @functools.partial(
        pl.kernel,
        mesh=_sc_mesh(),
        out_type=(
            jax.ShapeDtypeStruct((EP,), _f32),
            jax.ShapeDtypeStruct((NC, NPAD), _f32),
        ),
        scratch_types=[
            pltpu.VMEM((K,), _i32),
            pltpu.VMEM((K,), _i32),
            pltpu.VMEM((K, C), _f32),
            pltpu.VMEM((K, C), _f32),
            pltpu.VMEM((K, C), _f32),
            pltpu.VMEM((K, 16), _f32),
            pltpu.VMEM((K,), _f32),
            pltpu.VMEM((C,), _f32),
            pltpu.VMEM_SHARED((NPAD,), _f32),
            pltpu.SemaphoreType.DMA,
        ],
    )
    def sca(src_hbm, dst_hbm, xl_hbm, xr_hbm, ea_hbm, att_hbm, zn_hbm,
            ex_hbm, dp_hbm,
            src_v, dst_v, xlr, xrr, ear, accb, exb, attv, sden, sem):
        c = lax.axis_index("c")
        s = lax.axis_index("s")
        base_e = (c * NS + s) * EPW
        pltpu.sync_copy(att_hbm, attv)
        pltpu.sync_copy(zn_hbm.at[pl.ds(s * NSL, NSL)],
                        sden.at[pl.ds(s * NSL, NSL)])
        plsc.subcore_barrier()

        def chunk(ci, carry):
            base = base_e + ci * K
            pltpu.sync_copy(src_hbm.at[pl.ds(base, K)], src_v)
            pltpu.sync_copy(dst_hbm.at[pl.ds(base, K)], dst_v)
            pltpu.async_copy(xl_hbm.at[src_v], xlr, sem).wait()
            pltpu.async_copy(xr_hbm.at[dst_v], xrr, sem).wait()
            pltpu.sync_copy(ea_hbm.at[pl.ds(base, K)], ear)

            def edge(e, cc):
                acc = jnp.zeros((16,), _f32)
                for v in range(8):
                    sl = pl.ds(v * 16, 16)
                    m = xlr[e, sl] + xrr[e, sl] + ear[e, sl]
                    m = jnp.maximum(m, 0.0) + 0.2 * jnp.minimum(m, 0.0)
                    acc = acc + m * attv[sl]
                accb[e, :] = acc
                return cc

            lax.fori_loop(0, K, edge, 0)
            for g in range(8):
                rows = g * 16 + lax.iota(_i32, 16)
                ssum = jnp.zeros((16,), _f32)
                for l in range(16):
                    ssum = ssum + plsc.load_gather(
                        accb, [rows, jnp.full((16,), l, _i32)])
                exb[pl.ds(g * 16, 16)] = jnp.exp(ssum)
            pltpu.sync_copy(exb, ex_hbm.at[pl.ds(base, K)])
            pltpu.sync_copy(exb, sden.at[dst_v], add=True)
            return carry

        lax.fori_loop(0, NCH, chunk, 0)
        plsc.subcore_barrier()
        pltpu.sync_copy(sden.at[pl.ds(s * NSL, NSL)],
                        dp_hbm.at[c, pl.ds(s * NSL, NSL)])

    return sca(src, dst, xl, xr, ea, attf, zn)


def _sc_pass_b(src, dst, xl, ex, dp, zn, znc):
    @functools.partial(
        pl.kernel,
        mesh=_sc_mesh(),
        out_type=(
            jax.ShapeDtypeStruct((EP,), _f32),
            jax.ShapeDtypeStruct((NC, NPAD), _f32),
            jax.ShapeDtypeStruct((NC, NPAD, C), _f32),
        ),
        scratch_types=[
            pltpu.VMEM((NPAD,), _f32),
            pltpu.VMEM((NPAD,), _f32),
            pltpu.VMEM((K,), _i32),
            pltpu.VMEM((K,), _i32),
            pltpu.VMEM((K,), _f32),
            pltpu.VMEM((K,), _f32),
            pltpu.VMEM((K, C), _f32),
            pltpu.VMEM((K, C), _f32),
            pltpu.VMEM_SHARED((NPAD,), _f32),
            pltpu.VMEM_SHARED((NPAD, C), _f32),
            pltpu.SemaphoreType.DMA,
        ],
    )
    def scb(src_hbm, dst_hbm, xl_hbm, ex_hbm, dp_hbm, zn_hbm, znc_hbm,
            alpha_hbm, degp_hbm, hp_hbm,
            denv, tmpv, src_v, dst_v, exv, alv, xlr, scl, sdeg, sh, sem):
        c = lax.axis_index("c")
        s = lax.axis_index("s")
        base_e = (c * NS + s) * EPW
        pltpu.sync_copy(dp_hbm.at[0], denv)
        pltpu.sync_copy(dp_hbm.at[1], tmpv)

        def addb(i, cc):
            sl = pl.ds(i * 16, 16)
            denv[sl] = denv[sl] + tmpv[sl]
            return cc

        lax.fori_loop(0, NPAD // 16, addb, 0)
        pltpu.sync_copy(zn_hbm.at[pl.ds(s * NSL, NSL)],
                        sdeg.at[pl.ds(s * NSL, NSL)])
        pltpu.sync_copy(znc_hbm.at[pl.ds(s * NSL, NSL)],
                        sh.at[pl.ds(s * NSL, NSL)])
        plsc.subcore_barrier()

        def chunk(ci, carry):
            base = base_e + ci * K
            pltpu.sync_copy(src_hbm.at[pl.ds(base, K)], src_v)
            pltpu.sync_copy(dst_hbm.at[pl.ds(base, K)], dst_v)
            pltpu.sync_copy(ex_hbm.at[pl.ds(base, K)], exv)
            desc = pltpu.async_copy(xl_hbm.at[src_v], xlr, sem)
            for g in range(8):
                sl = pl.ds(g * 16, 16)
                dsum = plsc.load_gather(denv, [dst_v[sl]])
                alv[sl] = exv[sl] / (dsum + 1e-16)
            pltpu.sync_copy(alv, alpha_hbm.at[pl.ds(base, K)])
            pltpu.sync_copy(alv, sdeg.at[dst_v], add=True)
            desc.wait()

            def edge(e, cc):
                ab = plsc.load_gather(alv, [lax.broadcast(e, (16,))])
                for v in range(8):
                    sl = pl.ds(v * 16, 16)
                    scl[e, sl] = xlr[e, sl] * ab
                return cc

            lax.fori_loop(0, K, edge, 0)
            pltpu.sync_copy(scl, sh.at[dst_v], add=True)
            return carry

        lax.fori_loop(0, NCH, chunk, 0)
        plsc.subcore_barrier()
        pltpu.sync_copy(sdeg.at[pl.ds(s * NSL, NSL)],
                        degp_hbm.at[c, pl.ds(s * NSL, NSL)])
        pltpu.sync_copy(sh.at[pl.ds(s * NSL, NSL)],
                        hp_hbm.at[c, pl.ds(s * NSL, NSL)])

    return scb(src, dst, xl, ex, dp, zn, znc)


def _sc_pass_c(src, dst, hw, alpha, dis, znc):
    @functools.partial(
        pl.kernel,
        mesh=_sc_mesh(),
        out_type=jax.ShapeDtypeStruct((NC, NPAD, C), _f32),
        scratch_types=[
            pltpu.VMEM((NPAD,), _f32),
            pltpu.VMEM((K,), _i32),
            pltpu.VMEM((K,), _i32),
            pltpu.VMEM((K,), _f32),
            pltpu.VMEM((K,), _f32),
            pltpu.VMEM((K, C), _f32),
            pltpu.VMEM((K, C), _f32),
            pltpu.VMEM_SHARED((NPAD, C), _f32),
            pltpu.SemaphoreType.DMA,
        ],
    )
    def scc(src_hbm, dst_hbm, hw_hbm, al_hbm, dis_hbm, znc_hbm, h2p_hbm,
            disv, src_v, dst_v, alv, nrmv, hwr, scl, sh2, sem):
        c = lax.axis_index("c")
        s = lax.axis_index("s")
        base_e = (c * NS + s) * EPW
        pltpu.sync_copy(dis_hbm, disv)
        pltpu.sync_copy(znc_hbm.at[pl.ds(s * NSL, NSL)],
                        sh2.at[pl.ds(s * NSL, NSL)])
        plsc.subcore_barrier()

        def chunk(ci, carry):
            base = base_e + ci * K
            pltpu.sync_copy(src_hbm.at[pl.ds(base, K)], src_v)
            pltpu.sync_copy(dst_hbm.at[pl.ds(base, K)], dst_v)
            pltpu.sync_copy(al_hbm.at[pl.ds(base, K)], alv)
            desc = pltpu.async_copy(hw_hbm.at[src_v], hwr, sem)
            for g in range(8):
                sl = pl.ds(g * 16, 16)
                nrm = (plsc.load_gather(disv, [src_v[sl]]) * alv[sl]
                       * plsc.load_gather(disv, [dst_v[sl]]))
                nrmv[sl] = nrm
            desc.wait()

            def edge(e, cc):
                nb = plsc.load_gather(nrmv, [lax.broadcast(e, (16,))])
                for v in range(8):
                    sl = pl.ds(v * 16, 16)
                    scl[e, sl] = hwr[e, sl] * nb
                return cc

            lax.fori_loop(0, K, edge, 0)
            pltpu.sync_copy(scl, sh2.at[dst_v], add=True)
            return carry

        lax.fori_loop(0, NCH, chunk, 0)
        plsc.subcore_barrier()
        pltpu.sync_copy(sh2.at[pl.ds(s * NSL, NSL)],
                        h2p_hbm.at[c, pl.ds(s * NSL, NSL)])

    return scc(src, dst, hw, alpha, dis, znc)


def kernel(x, edge_index, edge_feature, W_l, W_r, W_e, att, b_gat,
           W_gcn, b_gcn, W_out, b_out):
    pad_e = EP - E
    src = jnp.concatenate([edge_index[0], jnp.zeros((pad_e,), _i32)])
    dst = jnp.concatenate([edge_index[1], jnp.full((pad_e,), N, _i32)])
    efp = jnp.concatenate(
        [edge_feature, jnp.zeros((pad_e, D_EDGE), _f32)], axis=0)
    attf = att.reshape(C)
    zn = jnp.zeros((NPAD,), _f32)
    znc = jnp.zeros((NPAD, C), _f32)

    xl, xr = _tc_xlxr(x, W_l, W_r)
    ea = _tc_ea(efp, W_e)
    ex, dp = _sc_pass_a(src, dst, xl, xr, ea, attf, zn)
    alpha_full, degp, hp = _sc_pass_b(src, dst, xl, ex, dp, zn, znc)
    hw, dis = _tc_mid(hp, b_gat, W_gcn, degp)
    h2p = _sc_pass_c(src, dst, hw, alpha_full, dis, znc)
    out_full = _tc_out(h2p, b_gcn, W_out, b_out)
    return (out_full[:N], alpha_full[:E].reshape(E, 1))


# speedup vs baseline: 4.3229x; 4.3229x over previous
import functools

import jax
import jax.numpy as jnp
from jax import lax
from jax.experimental import pallas as pl
from jax.experimental.pallas import tpu as pltpu
from jax.experimental.pallas import tpu_sc as plsc

N = 10000
E = 320000
C = 128
D_EDGE = 4
D_OUT = 2

NC = 2
NS = 16
NW = NC * NS
K = 128
NCH = 79
EP = NW * NCH * K
EPW = EP // NW
NPAD = 10240
NSL = NPAD // NS

_f32 = jnp.float32
_i32 = jnp.int32


def _tc_xlxr(x, W_l, W_r):
    def body(x_ref, wl_ref, wr_ref, xl_ref, xr_ref):
        xb = x_ref[...]
        xl_ref[...] = jnp.dot(xb, wl_ref[...], preferred_element_type=_f32)
        xr_ref[...] = jnp.dot(xb, wr_ref[...], preferred_element_type=_f32)

    return pl.pallas_call(
        body,
        grid=(10,),
        in_specs=[
            pl.BlockSpec((1000, C), lambda i: (i, 0)),
            pl.BlockSpec((C, C), lambda i: (0, 0)),
            pl.BlockSpec((C, C), lambda i: (0, 0)),
        ],
        out_specs=[
            pl.BlockSpec((1000, C), lambda i: (i, 0)),
            pl.BlockSpec((1000, C), lambda i: (i, 0)),
        ],
        out_shape=[
            jax.ShapeDtypeStruct((N, C), _f32),
            jax.ShapeDtypeStruct((N, C), _f32),
        ],
    )(x, W_l, W_r)


def _tc_ea(efp, W_e):
    def body(ef_ref, we_ref, ea_ref):
        ea_ref[...] = jnp.dot(ef_ref[...], we_ref[...],
                              preferred_element_type=_f32)

    return pl.pallas_call(
        body,
        grid=(NCH,),
        in_specs=[
            pl.BlockSpec((EP // NCH, D_EDGE), lambda i: (i, 0)),
            pl.BlockSpec((D_EDGE, C), lambda i: (0, 0)),
        ],
        out_specs=pl.BlockSpec((EP // NCH, C), lambda i: (i, 0)),
        out_shape=jax.ShapeDtypeStruct((EP, C), _f32),
    )(efp, W_e)


def _tc_mid(hp, b_gat, W_gcn, degp):
    def body(hp_ref, bg_ref, wg_ref, dp_ref, hw_ref, dis_ref):
        h = jnp.maximum(hp_ref[0] + hp_ref[1] + bg_ref[...][None, :], 0.0)
        hw_ref[...] = jnp.dot(h, wg_ref[...], preferred_element_type=_f32)
        deg = dp_ref[0] + dp_ref[1]
        dis_ref[...] = jnp.where(
            deg > 0, lax.rsqrt(jnp.maximum(deg, 1e-12)), 0.0)

    return pl.pallas_call(
        body,
        grid=(10,),
        in_specs=[
            pl.BlockSpec((2, 1024, C), lambda i: (0, i, 0)),
            pl.BlockSpec((C,), lambda i: (0,)),
            pl.BlockSpec((C, C), lambda i: (0, 0)),
            pl.BlockSpec((2, 1024), lambda i: (0, i)),
        ],
        out_specs=[
            pl.BlockSpec((1024, C), lambda i: (i, 0)),
            pl.BlockSpec((1024,), lambda i: (i,)),
        ],
        out_shape=[
            jax.ShapeDtypeStruct((NPAD, C), _f32),
            jax.ShapeDtypeStruct((NPAD,), _f32),
        ],
    )(hp, b_gat, W_gcn, degp)


def _tc_out(h2p, b_gcn, W_out, b_out):
    def body(h2p_ref, bg_ref, wo_ref, bo_ref, out_ref):
        h2 = jnp.maximum(h2p_ref[0] + h2p_ref[1] + bg_ref[...][None, :], 0.0)
        out_ref[...] = (jnp.dot(h2, wo_ref[...], preferred_element_type=_f32)
                        + bo_ref[...][None, :])

    return pl.pallas_call(
        body,
        grid=(10,),
        in_specs=[
            pl.BlockSpec((2, 1024, C), lambda i: (0, i, 0)),
            pl.BlockSpec((C,), lambda i: (0,)),
            pl.BlockSpec((C, D_OUT), lambda i: (0, 0)),
            pl.BlockSpec((D_OUT,), lambda i: (0,)),
        ],
        out_specs=pl.BlockSpec((1024, D_OUT), lambda i: (i, 0)),
        out_shape=jax.ShapeDtypeStruct((NPAD, D_OUT), _f32),
    )(h2p, b_gcn, W_out, b_out)


def _sc_mesh():
    return plsc.VectorSubcoreMesh(core_axis_name="c", subcore_axis_name="s",
                                  num_cores=NC, num_subcores=NS)


_SC_PARAMS = pltpu.CompilerParams(needs_layout_passes=False)


def _sc_pass_a(src, dst, xl, xr, ea, attf, zn):
    @functools.partial(
        pl.kernel,
        mesh=_sc_mesh(),
        compiler_params=_SC_PARAMS,
        out_type=(
            jax.ShapeDtypeStruct((EP,), _f32),
            jax.ShapeDtypeStruct((NC, NPAD), _f32),
        ),
        scratch_types=[
            pltpu.VMEM((K,), _i32),
            pltpu.VMEM((K,), _i32),
            pltpu.VMEM((K, C), _f32),
            pltpu.VMEM((K, C), _f32),
            pltpu.VMEM((K, C), _f32),
            pltpu.VMEM((K * 16,), _f32),
            pltpu.VMEM((K,), _f32),
            pltpu.VMEM((C,), _f32),
            pltpu.VMEM_SHARED((NPAD,), _f32),
            pltpu.SemaphoreType.DMA,
        ],
    )
    def sca(src_hbm, dst_hbm, xl_hbm, xr_hbm, ea_hbm, att_hbm, zn_hbm,
            ex_hbm, dp_hbm,
            src_v, dst_v, xlr, xrr, ear, accb, exb, attv, sden, sem):
        c = lax.axis_index("c")
        s = lax.axis_index("s")
        base_e = (c * NS + s) * EPW
        pltpu.sync_copy(att_hbm, attv)
        pltpu.sync_copy(zn_hbm.at[pl.ds(s * NSL, NSL)],
                        sden.at[pl.ds(s * NSL, NSL)])
        plsc.subcore_barrier()

        def chunk(ci, carry):
            base = base_e + ci * K
            pltpu.sync_copy(src_hbm.at[pl.ds(base, K)], src_v)
            pltpu.sync_copy(dst_hbm.at[pl.ds(base, K)], dst_v)
            pltpu.async_copy(xl_hbm.at[src_v], xlr, sem).wait()
            pltpu.async_copy(xr_hbm.at[dst_v], xrr, sem).wait()
            pltpu.sync_copy(ea_hbm.at[pl.ds(base, K)], ear)

            def edge(e, cc):
                acc = jnp.zeros((16,), _f32)
                for v in range(8):
                    sl = pl.ds(v * 16, 16)
                    m = xlr[e, sl] + xrr[e, sl] + ear[e, sl]
                    m = jnp.maximum(m, 0.0) + 0.2 * jnp.minimum(m, 0.0)
                    acc = acc + m * attv[sl]
                accb[pl.ds(e * 16, 16)] = acc
                return cc

            lax.fori_loop(0, K, edge, 0)
            for g in range(8):
                rows = (g * 256 + 16 * lax.iota(_i32, 16))
                ssum = jnp.zeros((16,), _f32)
                for l in range(16):
                    ssum = ssum + plsc.load_gather(accb, [rows + l])
                exb[pl.ds(g * 16, 16)] = jnp.exp(ssum)
            pltpu.sync_copy(exb, ex_hbm.at[pl.ds(base, K)])
            pltpu.sync_copy(exb, sden.at[dst_v], add=True)
            return carry

        lax.fori_loop(0, NCH, chunk, 0)
        plsc.subcore_barrier()
        pltpu.sync_copy(sden.at[pl.ds(s * NSL, NSL)],
                        dp_hbm.at[c, pl.ds(s * NSL, NSL)])

    return sca(src, dst, xl, xr, ea, attf, zn)


def _sc_pass_b(src, dst, xl, ex, dp, zn, znc):
    @functools.partial(
        pl.kernel,
        mesh=_sc_mesh(),
        compiler_params=_SC_PARAMS,
        out_type=(
            jax.ShapeDtypeStruct((EP,), _f32),
            jax.ShapeDtypeStruct((NC, NPAD), _f32),
            jax.ShapeDtypeStruct((NC, NPAD, C), _f32),
        ),
        scratch_types=[
            pltpu.VMEM((NPAD,), _f32),
            pltpu.VMEM((1024,), _f32),
            pltpu.VMEM((K,), _i32),
            pltpu.VMEM((K,), _i32),
            pltpu.VMEM((K,), _f32),
            pltpu.VMEM((K,), _f32),
            pltpu.VMEM((K, C), _f32),
            pltpu.VMEM((K, C), _f32),
            pltpu.VMEM_SHARED((NPAD,), _f32),
            pltpu.VMEM_SHARED((NPAD, C), _f32),
            pltpu.SemaphoreType.DMA,
        ],
    )
    def scb(src_hbm, dst_hbm, xl_hbm, ex_hbm, dp_hbm, zn_hbm, znc_hbm,
            alpha_hbm, degp_hbm, hp_hbm,
            denv, tmpv, src_v, dst_v, exv, alv, xlr, scl, sdeg, sh, sem):
        c = lax.axis_index("c")
        s = lax.axis_index("s")
        base_e = (c * NS + s) * EPW
        pltpu.sync_copy(dp_hbm.at[0], denv)

        def addblk(b, cc):
            pltpu.sync_copy(dp_hbm.at[1, pl.ds(b * 1024, 1024)], tmpv)

            def addb(i, c2):
                dsl = pl.ds(b * 1024 + i * 16, 16)
                denv[dsl] = denv[dsl] + tmpv[pl.ds(i * 16, 16)]
                return c2

            lax.fori_loop(0, 64, addb, 0)
            return cc

        lax.fori_loop(0, NPAD // 1024, addblk, 0)
        pltpu.sync_copy(zn_hbm.at[pl.ds(s * NSL, NSL)],
                        sdeg.at[pl.ds(s * NSL, NSL)])
        pltpu.sync_copy(znc_hbm.at[pl.ds(s * NSL, NSL)],
                        sh.at[pl.ds(s * NSL, NSL)])
        plsc.subcore_barrier()

        def chunk(ci, carry):
            base = base_e + ci * K
            pltpu.sync_copy(src_hbm.at[pl.ds(base, K)], src_v)
            pltpu.sync_copy(dst_hbm.at[pl.ds(base, K)], dst_v)
            pltpu.sync_copy(ex_hbm.at[pl.ds(base, K)], exv)
            desc = pltpu.async_copy(xl_hbm.at[src_v], xlr, sem)
            for g in range(8):
                sl = pl.ds(g * 16, 16)
                dsum = plsc.load_gather(denv, [dst_v[sl]])
                alv[sl] = exv[sl] / (dsum + 1e-16)
            pltpu.sync_copy(alv, alpha_hbm.at[pl.ds(base, K)])
            pltpu.sync_copy(alv, sdeg.at[dst_v], add=True)
            desc.wait()

            def edge(e, cc):
                ab = plsc.load_gather(alv, [lax.broadcast(e, (16,))])
                for v in range(8):
                    sl = pl.ds(v * 16, 16)
                    scl[e, sl] = xlr[e, sl] * ab
                return cc

            lax.fori_loop(0, K, edge, 0)
            pltpu.sync_copy(scl, sh.at[dst_v], add=True)
            return carry

        lax.fori_loop(0, NCH, chunk, 0)
        plsc.subcore_barrier()
        pltpu.sync_copy(sdeg.at[pl.ds(s * NSL, NSL)],
                        degp_hbm.at[c, pl.ds(s * NSL, NSL)])
        pltpu.sync_copy(sh.at[pl.ds(s * NSL, NSL)],
                        hp_hbm.at[c, pl.ds(s * NSL, NSL)])

    return scb(src, dst, xl, ex, dp, zn, znc)


def _sc_pass_c(src, dst, hw, alpha, dis, znc):
    @functools.partial(
        pl.kernel,
        mesh=_sc_mesh(),
        compiler_params=_SC_PARAMS,
        out_type=jax.ShapeDtypeStruct((NC, NPAD, C), _f32),
        scratch_types=[
            pltpu.VMEM((NPAD,), _f32),
            pltpu.VMEM((K,), _i32),
            pltpu.VMEM((K,), _i32),
            pltpu.VMEM((K,), _f32),
            pltpu.VMEM((K,), _f32),
            pltpu.VMEM((K, C), _f32),
            pltpu.VMEM((K, C), _f32),
            pltpu.VMEM_SHARED((NPAD, C), _f32),
            pltpu.SemaphoreType.DMA,
        ],
    )
    def scc(src_hbm, dst_hbm, hw_hbm, al_hbm, dis_hbm, znc_hbm, h2p_hbm,
            disv, src_v, dst_v, alv, nrmv, hwr, scl, sh2, sem):
        c = lax.axis_index("c")
        s = lax.axis_index("s")
        base_e = (c * NS + s) * EPW
        pltpu.sync_copy(dis_hbm, disv)
        pltpu.sync_copy(znc_hbm.at[pl.ds(s * NSL, NSL)],
                        sh2.at[pl.ds(s * NSL, NSL)])
        plsc.subcore_barrier()

        def chunk(ci, carry):
            base = base_e + ci * K
            pltpu.sync_copy(src_hbm.at[pl.ds(base, K)], src_v)
            pltpu.sync_copy(dst_hbm.at[pl.ds(base, K)], dst_v)
            pltpu.sync_copy(al_hbm.at[pl.ds(base, K)], alv)
            desc = pltpu.async_copy(hw_hbm.at[src_v], hwr, sem)
            for g in range(8):
                sl = pl.ds(g * 16, 16)
                nrm = (plsc.load_gather(disv, [src_v[sl]]) * alv[sl]
                       * plsc.load_gather(disv, [dst_v[sl]]))
                nrmv[sl] = nrm
            desc.wait()

            def edge(e, cc):
                nb = plsc.load_gather(nrmv, [lax.broadcast(e, (16,))])
                for v in range(8):
                    sl = pl.ds(v * 16, 16)
                    scl[e, sl] = hwr[e, sl] * nb
                return cc

            lax.fori_loop(0, K, edge, 0)
            pltpu.sync_copy(scl, sh2.at[dst_v], add=True)
            return carry

        lax.fori_loop(0, NCH, chunk, 0)
        plsc.subcore_barrier()
        pltpu.sync_copy(sh2.at[pl.ds(s * NSL, NSL)],
                        h2p_hbm.at[c, pl.ds(s * NSL, NSL)])

    return scc(src, dst, hw, alpha, dis, znc)


def kernel(x, edge_index, edge_feature, W_l, W_r, W_e, att, b_gat,
           W_gcn, b_gcn, W_out, b_out):
    pad_e = EP - E
    src = jnp.concatenate([edge_index[0], jnp.zeros((pad_e,), _i32)])
    dst = jnp.concatenate([edge_index[1], jnp.full((pad_e,), N, _i32)])
    efp = jnp.concatenate(
        [edge_feature, jnp.zeros((pad_e, D_EDGE), _f32)], axis=0)
    attf = att.reshape(C)
    zn = jnp.zeros((NPAD,), _f32)
    znc = jnp.zeros((NPAD, C), _f32)

    xl, xr = _tc_xlxr(x, W_l, W_r)
    ea = _tc_ea(efp, W_e)
    ex, dp = _sc_pass_a(src, dst, xl, xr, ea, attf, zn)
    alpha_full, degp, hp = _sc_pass_b(src, dst, xl, ex, dp, zn, znc)
    hw, dis = _tc_mid(hp, b_gat, W_gcn, degp)
    h2p = _sc_pass_c(src, dst, hw, alpha_full, dis, znc)
    out_full = _tc_out(h2p, b_gcn, W_out, b_out)
    return (out_full[:N], alpha_full[:E].reshape(E, 1))
